# Initial kernel scaffold; baseline (speedup 1.0000x reference)
import jax
import jax.numpy as jnp
from jax.experimental import pallas as pl


def kernel(x, edge_index, batch, W1, b1, W2, b2, Wf1, bf1, Wf2, bf2):
    raise NotImplementedError("write your pallas kernel here")


# speedup vs baseline: 22.9425x; 22.9425x over previous
import functools
import jax
import jax.numpy as jnp
from jax import lax
from jax.experimental import pallas as pl
from jax.experimental.pallas import tpu as pltpu
from jax.experimental.pallas import tpu_sc as plsc

N = 10000
G = 64
NC, NS, L = 2, 16, 16
NW = NC * NS
CHUNK = 128
N_PAD = 10112
RPT = N_PAD // NS

_SC_MESH = plsc.VectorSubcoreMesh(core_axis_name="c", subcore_axis_name="s")
_SC_PARAMS = pltpu.CompilerParams(needs_layout_passes=False,
                                  use_tc_tiling_on_sc=False)


def _deg_body(dst_hbm, out_hbm, dstbuf, hist):
    c = lax.axis_index("c")
    s = lax.axis_index("s")
    wid = s * NC + c
    ept = dstbuf.shape[0]

    zeros = jnp.zeros((L,), jnp.float32)

    def zero(i, carry):
        hist[pl.ds(pl.multiple_of(i * L, L), L)] = zeros
        return carry

    lax.fori_loop(0, N_PAD // L, zero, 0)

    pltpu.sync_copy(dst_hbm.at[wid], dstbuf)

    ones = jnp.full((L,), 1.0, jnp.float32)

    def body(i, carry):
        idx = dstbuf[pl.ds(pl.multiple_of(i * L, L), L)]
        plsc.addupdate_scatter(hist, [idx], ones)
        return carry

    lax.fori_loop(0, ept // L, body, 0)
    pltpu.sync_copy(hist, out_hbm.at[wid])


def _make_deg_call(ept):
    return pl.kernel(
        _deg_body,
        out_type=jax.ShapeDtypeStruct((NW, N_PAD), jnp.float32),
        mesh=_SC_MESH,
        compiler_params=_SC_PARAMS,
        scratch_types=[
            pltpu.VMEM((ept,), jnp.int32),
            pltpu.VMEM((N_PAD,), jnp.float32),
        ],
    )


def _edge_body(y_hbm, src_hbm, dst_hbm, z_hbm, out_hbm, sbuf, dbuf, rows, acc,
               gsem):
    c = lax.axis_index("c")
    s = lax.axis_index("s")
    wid = s * NC + c
    nch = sbuf.shape[0]

    pltpu.sync_copy(z_hbm, acc.at[pl.ds(s * RPT, RPT)])
    pltpu.sync_copy(src_hbm.at[wid], sbuf)
    pltpu.sync_copy(dst_hbm.at[wid], dbuf)
    plsc.subcore_barrier()

    def body(j, carry):
        pltpu.async_copy(y_hbm.at[sbuf.at[j]], rows, gsem).wait()
        pltpu.sync_copy(rows, acc.at[dbuf.at[j]], add=True)
        return carry

    lax.fori_loop(0, nch, body, 0)

    plsc.subcore_barrier()
    pltpu.sync_copy(acc.at[pl.ds(s * RPT, RPT)],
                    out_hbm.at[c].at[pl.ds(s * RPT, RPT)])


def _make_edge_call(nch, feat):
    return pl.kernel(
        _edge_body,
        out_type=jax.ShapeDtypeStruct((NC, N_PAD, feat), jnp.float32),
        mesh=_SC_MESH,
        compiler_params=_SC_PARAMS,
        scratch_types=[
            pltpu.VMEM((nch, CHUNK), jnp.int32),
            pltpu.VMEM((nch, CHUNK), jnp.int32),
            pltpu.VMEM((CHUNK, feat), jnp.float32),
            pltpu.VMEM_SHARED((N_PAD, feat), jnp.float32),
            pltpu.SemaphoreType.DMA,
        ],
    )


def _tc1_body(x_ref, w_ref, degp_ref, y_ref, dinv_ref):
    degp = degp_ref[...]
    ones = jnp.ones((NW, 1), jnp.float32)
    deg = lax.dot_general(degp, ones, (((0,), (0,)), ((), ())),
                          precision=lax.Precision.HIGHEST) + 1.0
    dinv = 1.0 / jnp.sqrt(deg)
    xw = jnp.dot(x_ref[...], w_ref[...])
    y_ref[...] = xw * dinv
    dinv_ref[...] = dinv


def _tc2_body(acc_ref, y_ref, dinv_ref, b_ref, w_ref, x1_ref, y2_ref):
    dinv = dinv_ref[...]
    tot = acc_ref[0] + acc_ref[1] + y_ref[...]
    x1 = jax.nn.relu(tot * dinv + b_ref[...])
    x1_ref[...] = x1
    y2_ref[...] = jnp.dot(x1, w_ref[...]) * dinv


def _tc3_body(acc_ref, y_ref, dinv_ref, b_ref, x1_ref, batch_ref,
              wf1_ref, bf1_ref, wf2_ref, bf2_ref, out_ref):
    dinv = dinv_ref[...]
    tot = acc_ref[0] + acc_ref[1] + y_ref[...]
    x2 = jax.nn.relu(tot * dinv + b_ref[...])
    xj = jnp.concatenate([x1_ref[:N], x2[:N]], axis=1)
    gids = lax.broadcasted_iota(jnp.int32, (G, N), 0)
    m = (batch_ref[...] == gids).astype(jnp.float32)
    sums = jnp.dot(m, xj, precision=lax.Precision.HIGHEST)
    counts = jnp.sum(m, axis=1, keepdims=True)
    pool = sums / jnp.maximum(counts, 1.0)
    h = jax.nn.relu(jnp.dot(pool, wf1_ref[...]) + bf1_ref[...])
    out_ref[...] = jnp.dot(h, wf2_ref[...]) + bf2_ref[...]


def _tc_call(body, out_shapes):
    return pl.pallas_call(body, out_shape=out_shapes)


def kernel(x, edge_index, batch, W1, b1, W2, b2, Wf1, bf1, Wf2, bf2):
    E = edge_index.shape[1]
    ept = -(-E // NW)
    nch = -(-ept // CHUNK)
    e_pad = NW * nch * CHUNK

    src = jnp.concatenate(
        [edge_index[0], jnp.zeros((e_pad - E,), jnp.int32)]
    ).reshape(NW, nch, CHUNK)
    dst = jnp.concatenate(
        [edge_index[1], jnp.full((e_pad - E,), N, jnp.int32)]
    ).reshape(NW, nch, CHUNK)

    x_pad = jnp.pad(x, ((0, N_PAD - N), (0, 0)))
    z32 = jnp.zeros((RPT, 32), jnp.float32)
    z64 = jnp.zeros((RPT, 64), jnp.float32)

    deg_parts = _make_deg_call(nch * CHUNK)(dst.reshape(NW, nch * CHUNK))

    y1, dinv = _tc_call(_tc1_body, (
        jax.ShapeDtypeStruct((N_PAD, 32), jnp.float32),
        jax.ShapeDtypeStruct((N_PAD, 1), jnp.float32),
    ))(x_pad, W1, deg_parts)

    acc1 = _make_edge_call(nch, 32)(y1, src, dst, z32)

    x1, y2 = _tc_call(_tc2_body, (
        jax.ShapeDtypeStruct((N_PAD, 32), jnp.float32),
        jax.ShapeDtypeStruct((N_PAD, 64), jnp.float32),
    ))(acc1, y1, dinv, b1.reshape(1, 32), W2)

    acc2 = _make_edge_call(nch, 64)(y2, src, dst, z64)

    out = _tc_call(_tc3_body, (
        jax.ShapeDtypeStruct((G, 1), jnp.float32),
    ))(acc2, y2, dinv, b2.reshape(1, 64), x1, batch.reshape(1, N),
       Wf1, bf1.reshape(1, 128), Wf2, bf2.reshape(1, 1))[0]

    return out.reshape(-1)


# speedup vs baseline: 24.2644x; 1.0576x over previous
import functools
import jax
import jax.numpy as jnp
from jax import lax
from jax.experimental import pallas as pl
from jax.experimental.pallas import tpu as pltpu
from jax.experimental.pallas import tpu_sc as plsc

N = 10000
G = 64
NC, NS, L = 2, 16, 16
NW = NC * NS
CHUNK = 128
N_PAD = 10112
RPT = N_PAD // NS

_SC_MESH = plsc.VectorSubcoreMesh(core_axis_name="c", subcore_axis_name="s")
_SC_PARAMS = pltpu.CompilerParams(needs_layout_passes=False,
                                  use_tc_tiling_on_sc=False)


def _deg_body(dst_hbm, out_hbm, dstbuf, hist):
    c = lax.axis_index("c")
    s = lax.axis_index("s")
    wid = s * NC + c
    ept = dstbuf.shape[0]

    zeros = jnp.zeros((L,), jnp.float32)

    def zero(i, carry):
        hist[pl.ds(pl.multiple_of(i * L, L), L)] = zeros
        return carry

    lax.fori_loop(0, N_PAD // L, zero, 0, unroll=8)

    pltpu.sync_copy(dst_hbm.at[wid], dstbuf)

    ones = jnp.full((L,), 1.0, jnp.float32)

    def body(i, carry):
        idx = dstbuf[pl.ds(pl.multiple_of(i * L, L), L)]
        plsc.addupdate_scatter(hist, [idx], ones)
        return carry

    lax.fori_loop(0, ept // L, body, 0, unroll=8)
    pltpu.sync_copy(hist, out_hbm.at[wid])


def _make_deg_call(ept):
    return pl.kernel(
        _deg_body,
        out_type=jax.ShapeDtypeStruct((NW, N_PAD), jnp.float32),
        mesh=_SC_MESH,
        compiler_params=_SC_PARAMS,
        scratch_types=[
            pltpu.VMEM((ept,), jnp.int32),
            pltpu.VMEM((N_PAD,), jnp.float32),
        ],
    )


def _edge_body(y_hbm, src_hbm, dst_hbm, z_hbm, out_hbm, sbuf, dbuf,
               r0, r1, r2, r3, acc, g0, g1, g2, g3, s0, s1, s2, s3):
    c = lax.axis_index("c")
    s = lax.axis_index("s")
    wid = s * NC + c
    nch = sbuf.shape[0]
    rows = (r0, r1, r2, r3)
    gsem = (g0, g1, g2, g3)
    ssem = (s0, s1, s2, s3)

    def gather(j, b):
        return pltpu.make_async_copy(y_hbm.at[sbuf.at[j]], rows[b], gsem[b])

    def scat(j, b):
        return pltpu.make_async_copy(rows[b], acc.at[dbuf.at[j]], ssem[b])

    pltpu.sync_copy(z_hbm, acc.at[pl.ds(s * RPT, RPT)])
    pltpu.sync_copy(src_hbm.at[wid], sbuf)
    pltpu.sync_copy(dst_hbm.at[wid], dbuf)
    plsc.subcore_barrier()

    gather(0, 0).start()
    gather(1, 1).start()

    def grp(g, carry):
        for b in range(4):
            j = g * 4 + b
            bn = (b + 2) % 4

            @pl.when(j >= 2)
            def _():
                scat(j - 2, bn).wait()

            @pl.when(j + 2 < nch)
            def _():
                gather(j + 2, bn).start()

            gather(j, b).wait()
            scat(j, b).start(add=True)
        return carry

    lax.fori_loop(0, nch // 4, grp, 0)
    scat(nch - 2, (nch - 2) % 4).wait()
    scat(nch - 1, (nch - 1) % 4).wait()

    plsc.subcore_barrier()
    pltpu.sync_copy(acc.at[pl.ds(s * RPT, RPT)],
                    out_hbm.at[c].at[pl.ds(s * RPT, RPT)])


def _make_edge_call(nch, feat):
    return pl.kernel(
        _edge_body,
        out_type=jax.ShapeDtypeStruct((NC, N_PAD, feat), jnp.float32),
        mesh=_SC_MESH,
        compiler_params=_SC_PARAMS,
        scratch_types=[
            pltpu.VMEM((nch, CHUNK), jnp.int32),
            pltpu.VMEM((nch, CHUNK), jnp.int32),
        ] + [pltpu.VMEM((CHUNK, feat), jnp.float32)] * 4 + [
            pltpu.VMEM_SHARED((N_PAD, feat), jnp.float32),
        ] + [pltpu.SemaphoreType.DMA] * 8,
    )


def _tc1_body(x_ref, w_ref, degp_ref, y_ref, dinv_ref):
    degp = degp_ref[...]
    ones = jnp.ones((NW, 1), jnp.float32)
    deg = lax.dot_general(degp, ones, (((0,), (0,)), ((), ())),
                          precision=lax.Precision.HIGHEST) + 1.0
    dinv = 1.0 / jnp.sqrt(deg)
    xw = jnp.dot(x_ref[...], w_ref[...])
    y_ref[...] = xw * dinv
    dinv_ref[...] = dinv


def _tc2_body(acc_ref, y_ref, dinv_ref, b_ref, w_ref, x1_ref, y2_ref):
    dinv = dinv_ref[...]
    tot = acc_ref[0] + acc_ref[1] + y_ref[...]
    x1 = jax.nn.relu(tot * dinv + b_ref[...])
    x1_ref[...] = x1
    y2_ref[...] = jnp.dot(x1, w_ref[...]) * dinv


def _tc3_body(acc_ref, y_ref, dinv_ref, b_ref, x1_ref, batch_ref,
              wf1_ref, bf1_ref, wf2_ref, bf2_ref, out_ref):
    dinv = dinv_ref[...]
    tot = acc_ref[0] + acc_ref[1] + y_ref[...]
    x2 = jax.nn.relu(tot * dinv + b_ref[...])
    xj = jnp.concatenate([x1_ref[:N], x2[:N]], axis=1)
    gids = lax.broadcasted_iota(jnp.int32, (G, N), 0)
    m = (batch_ref[...] == gids).astype(jnp.float32)
    sums = jnp.dot(m, xj, precision=lax.Precision.HIGHEST)
    counts = jnp.sum(m, axis=1, keepdims=True)
    pool = sums / jnp.maximum(counts, 1.0)
    h = jax.nn.relu(jnp.dot(pool, wf1_ref[...]) + bf1_ref[...])
    out_ref[...] = jnp.dot(h, wf2_ref[...]) + bf2_ref[...]


def _tc_call(body, out_shapes):
    return pl.pallas_call(body, out_shape=out_shapes)


def kernel(x, edge_index, batch, W1, b1, W2, b2, Wf1, bf1, Wf2, bf2):
    E = edge_index.shape[1]
    ept = -(-E // NW)
    nch = -(-(-(-ept // CHUNK)) // 4) * 4
    e_pad = NW * nch * CHUNK

    src = jnp.concatenate(
        [edge_index[0], jnp.zeros((e_pad - E,), jnp.int32)]
    ).reshape(NW, nch, CHUNK)
    dst = jnp.concatenate(
        [edge_index[1], jnp.full((e_pad - E,), N, jnp.int32)]
    ).reshape(NW, nch, CHUNK)

    x_pad = jnp.pad(x, ((0, N_PAD - N), (0, 0)))
    z32 = jnp.zeros((RPT, 32), jnp.float32)
    z64 = jnp.zeros((RPT, 64), jnp.float32)

    deg_parts = _make_deg_call(nch * CHUNK)(dst.reshape(NW, nch * CHUNK))

    y1, dinv = _tc_call(_tc1_body, (
        jax.ShapeDtypeStruct((N_PAD, 32), jnp.float32),
        jax.ShapeDtypeStruct((N_PAD, 1), jnp.float32),
    ))(x_pad, W1, deg_parts)

    acc1 = _make_edge_call(nch, 32)(y1, src, dst, z32)

    x1, y2 = _tc_call(_tc2_body, (
        jax.ShapeDtypeStruct((N_PAD, 32), jnp.float32),
        jax.ShapeDtypeStruct((N_PAD, 64), jnp.float32),
    ))(acc1, y1, dinv, b1.reshape(1, 32), W2)

    acc2 = _make_edge_call(nch, 64)(y2, src, dst, z64)

    out = _tc_call(_tc3_body, (
        jax.ShapeDtypeStruct((G, 1), jnp.float32),
    ))(acc2, y2, dinv, b2.reshape(1, 64), x1, batch.reshape(1, N),
       Wf1, bf1.reshape(1, 128), Wf2, bf2.reshape(1, 1))[0]

    return out.reshape(-1)


# speedup vs baseline: 24.5181x; 1.0105x over previous
import functools
import jax
import jax.numpy as jnp
from jax import lax
from jax.experimental import pallas as pl
from jax.experimental.pallas import tpu as pltpu
from jax.experimental.pallas import tpu_sc as plsc

N = 10000
G = 64
NC, NS, L = 2, 16, 16
NW = NC * NS
CHUNK = 128
N_PAD = 10112
RPT = N_PAD // NS

_SC_MESH = plsc.VectorSubcoreMesh(core_axis_name="c", subcore_axis_name="s")
_SC_PARAMS = pltpu.CompilerParams(needs_layout_passes=False,
                                  use_tc_tiling_on_sc=False)


def _deg_body(dst_hbm, out_hbm, dstbuf, hist):
    c = lax.axis_index("c")
    s = lax.axis_index("s")
    wid = s * NC + c
    ept = dstbuf.shape[0]

    zeros = jnp.zeros((L,), jnp.float32)

    def zero(i, carry):
        hist[pl.ds(pl.multiple_of(i * L, L), L)] = zeros
        return carry

    lax.fori_loop(0, N_PAD // L, zero, 0, unroll=8)

    pltpu.sync_copy(dst_hbm.at[wid], dstbuf)

    ones = jnp.full((L,), 1.0, jnp.float32)

    def body(i, carry):
        idx = dstbuf[pl.ds(pl.multiple_of(i * L, L), L)]
        plsc.addupdate_scatter(hist, [idx], ones)
        return carry

    lax.fori_loop(0, ept // L, body, 0, unroll=8)
    pltpu.sync_copy(hist, out_hbm.at[wid])


def _make_deg_call(ept):
    return pl.kernel(
        _deg_body,
        out_type=jax.ShapeDtypeStruct((NW, N_PAD), jnp.float32),
        mesh=_SC_MESH,
        compiler_params=_SC_PARAMS,
        scratch_types=[
            pltpu.VMEM((ept,), jnp.int32),
            pltpu.VMEM((N_PAD,), jnp.float32),
        ],
    )


NBUF = 8
LA = 6


def _edge_body(y_hbm, src_hbm, dst_hbm, z_hbm, out_hbm, sbuf, dbuf,
               rows, acc, gsem, ssem):
    c = lax.axis_index("c")
    s = lax.axis_index("s")
    wid = s * NC + c
    nch = sbuf.shape[0]

    def gather(j, b):
        return pltpu.make_async_copy(y_hbm.at[sbuf.at[j]], rows[b], gsem[b])

    def scat(j, b):
        return pltpu.make_async_copy(rows[b], acc.at[dbuf.at[j]], ssem[b])

    pltpu.sync_copy(z_hbm, acc.at[pl.ds(s * RPT, RPT)])
    pltpu.sync_copy(src_hbm.at[wid], sbuf)
    pltpu.sync_copy(dst_hbm.at[wid], dbuf)
    plsc.subcore_barrier()

    for b in range(LA):
        gather(b, b).start()

    def grp(g, carry):
        for b in range(NBUF):
            j = g * NBUF + b
            bn = (b + LA) % NBUF

            @pl.when(j + LA < nch)
            def _():
                @pl.when(j >= NBUF - LA)
                def _():
                    scat(j - (NBUF - LA), bn).wait()

                gather(j + LA, bn).start()

            gather(j, b).wait()
            scat(j, b).start(add=True)
        return carry

    lax.fori_loop(0, nch // NBUF, grp, 0)
    for k in range(NBUF):
        c0 = nch - NBUF + k
        scat(c0, c0 % NBUF).wait()

    plsc.subcore_barrier()
    pltpu.sync_copy(acc.at[pl.ds(s * RPT, RPT)],
                    out_hbm.at[c].at[pl.ds(s * RPT, RPT)])


def _make_edge_call(nch, feat):
    return pl.kernel(
        _edge_body,
        out_type=jax.ShapeDtypeStruct((NC, N_PAD, feat), jnp.float32),
        mesh=_SC_MESH,
        compiler_params=_SC_PARAMS,
        scratch_types=[
            pltpu.VMEM((nch, CHUNK), jnp.int32),
            pltpu.VMEM((nch, CHUNK), jnp.int32),
            tuple(pltpu.VMEM((CHUNK, feat), jnp.float32)
                  for _ in range(NBUF)),
            pltpu.VMEM_SHARED((N_PAD, feat), jnp.float32),
            tuple(pltpu.SemaphoreType.DMA for _ in range(NBUF)),
            tuple(pltpu.SemaphoreType.DMA for _ in range(NBUF)),
        ],
    )


def _tc1_body(x_ref, w_ref, degp_ref, y_ref, dinv_ref):
    degp = degp_ref[...]
    ones = jnp.ones((NW, 1), jnp.float32)
    deg = lax.dot_general(degp, ones, (((0,), (0,)), ((), ())),
                          precision=lax.Precision.HIGHEST) + 1.0
    dinv = 1.0 / jnp.sqrt(deg)
    xw = jnp.dot(x_ref[...], w_ref[...])
    y_ref[...] = xw * dinv
    dinv_ref[...] = dinv


def _tc2_body(acc_ref, y_ref, dinv_ref, b_ref, w_ref, x1_ref, y2_ref):
    dinv = dinv_ref[...]
    tot = acc_ref[0] + acc_ref[1] + y_ref[...]
    x1 = jax.nn.relu(tot * dinv + b_ref[...])
    x1_ref[...] = x1
    y2_ref[...] = jnp.dot(x1, w_ref[...]) * dinv


def _tc3_body(acc_ref, y_ref, dinv_ref, b_ref, x1_ref, batch_ref,
              wf1_ref, bf1_ref, wf2_ref, bf2_ref, out_ref):
    dinv = dinv_ref[...]
    tot = acc_ref[0] + acc_ref[1] + y_ref[...]
    x2 = jax.nn.relu(tot * dinv + b_ref[...])
    xj = jnp.concatenate([x1_ref[:N], x2[:N]], axis=1)
    gids = lax.broadcasted_iota(jnp.int32, (G, N), 0)
    m = (batch_ref[...] == gids).astype(jnp.float32)
    sums = jnp.dot(m, xj, precision=lax.Precision.HIGHEST)
    counts = jnp.sum(m, axis=1, keepdims=True)
    pool = sums / jnp.maximum(counts, 1.0)
    h = jax.nn.relu(jnp.dot(pool, wf1_ref[...]) + bf1_ref[...])
    out_ref[...] = jnp.dot(h, wf2_ref[...]) + bf2_ref[...]


def _tc_call(body, out_shapes):
    return pl.pallas_call(body, out_shape=out_shapes)


def kernel(x, edge_index, batch, W1, b1, W2, b2, Wf1, bf1, Wf2, bf2):
    E = edge_index.shape[1]
    ept = -(-E // NW)
    nch = -(-(-(-ept // CHUNK)) // NBUF) * NBUF
    e_pad = NW * nch * CHUNK

    src = jnp.concatenate(
        [edge_index[0], jnp.zeros((e_pad - E,), jnp.int32)]
    ).reshape(NW, nch, CHUNK)
    dst = jnp.concatenate(
        [edge_index[1], jnp.full((e_pad - E,), N, jnp.int32)]
    ).reshape(NW, nch, CHUNK)

    x_pad = jnp.pad(x, ((0, N_PAD - N), (0, 0)))
    z32 = jnp.zeros((RPT, 32), jnp.float32)
    z64 = jnp.zeros((RPT, 64), jnp.float32)

    deg_parts = _make_deg_call(nch * CHUNK)(dst.reshape(NW, nch * CHUNK))

    y1, dinv = _tc_call(_tc1_body, (
        jax.ShapeDtypeStruct((N_PAD, 32), jnp.float32),
        jax.ShapeDtypeStruct((N_PAD, 1), jnp.float32),
    ))(x_pad, W1, deg_parts)

    acc1 = _make_edge_call(nch, 32)(y1, src, dst, z32)

    x1, y2 = _tc_call(_tc2_body, (
        jax.ShapeDtypeStruct((N_PAD, 32), jnp.float32),
        jax.ShapeDtypeStruct((N_PAD, 64), jnp.float32),
    ))(acc1, y1, dinv, b1.reshape(1, 32), W2)

    acc2 = _make_edge_call(nch, 64)(y2, src, dst, z64)

    out = _tc_call(_tc3_body, (
        jax.ShapeDtypeStruct((G, 1), jnp.float32),
    ))(acc2, y2, dinv, b2.reshape(1, 64), x1, batch.reshape(1, N),
       Wf1, bf1.reshape(1, 128), Wf2, bf2.reshape(1, 1))[0]

    return out.reshape(-1)
